# Initial kernel scaffold; baseline (speedup 1.0000x reference)
import jax
import jax.numpy as jnp
from jax.experimental import pallas as pl


def kernel(x, tables):
    raise NotImplementedError("write your pallas kernel here")


# speedup vs baseline: 2.0082x; 2.0082x over previous
import jax
import jax.numpy as jnp
import numpy as np
from jax import lax
from jax.experimental import pallas as pl
from jax.experimental.pallas import tpu as pltpu
from jax.experimental.pallas import tpu_sc as plsc

_N_POINTS = 524288
_N_LEVELS = 16
_N_FEAT = 2
_LOG2_HASH = 19
_HSIZE = 1 << _LOG2_HASH
_MASK = _HSIZE - 1
_GROWTH = np.exp((np.log(512.0) - np.log(16.0)) / (_N_LEVELS - 1))
_RESOLUTIONS = [int(np.ceil(16 * _GROWTH ** i)) for i in range(_N_LEVELS)]
_P1 = np.int32(2654435761 - (1 << 32))
_P2 = np.int32(805459861)

_NC, _NS = 2, 16
_NW = _NC * _NS
_PTS_W = _N_POINTS // _NW
_CH = 1024
_NCHUNK = _PTS_W // _CH
_G = _CH // 16
_NROW = _CH * 8 // 128


def _body(xt, tbl, out, xb, wb, idxb, rows, ob, sem):
    i32 = jnp.int32
    wid = lax.axis_index("s") * i32(_NC) + lax.axis_index("c")
    base = wid * i32(_PTS_W)
    iota = lax.iota(jnp.int32, 16)
    feat0 = jnp.zeros((16,), jnp.int32)
    feat1 = jnp.ones((16,), jnp.int32)

    def chunk_body(ch, carry):
        cbase = base + ch * i32(_CH)
        for d in range(3):
            pltpu.sync_copy(xt.at[pl.ds(i32(d * _N_POINTS) + cbase, _CH)],
                            xb.at[pl.ds(i32(d * _CH), _CH)])

        def clip_body(g, c):
            p = g * i32(16)
            for d in range(3):
                v = xb[pl.ds(i32(d * _CH) + p, 16)]
                xb[pl.ds(i32(d * _CH) + p, 16)] = jnp.minimum(
                    jnp.maximum(v, 0.0), 1.0 - 1e-6)
            return c

        lax.fori_loop(jnp.int32(0), jnp.int32(_G), clip_body, 0)

        for lvl in range(_N_LEVELS):
            res = jnp.float32(_RESOLUTIONS[lvl])
            lvl_off = jnp.int32(lvl << _LOG2_HASH)

            def pass_a(g, c, lvl_off=lvl_off, res=res):
                p = g * i32(16)
                r0 = g >> i32(3)
                q = (g & i32(7)) * i32(16)
                sx = xb[pl.ds(i32(0) + p, 16)] * res
                sy = xb[pl.ds(i32(1024) + p, 16)] * res
                sz = xb[pl.ds(i32(2048) + p, 16)] * res
                ix = sx.astype(jnp.int32)
                iy = sy.astype(jnp.int32)
                iz = sz.astype(jnp.int32)
                wb[pl.ds(i32(0) + p, 16)] = sx - ix.astype(jnp.float32)
                wb[pl.ds(i32(1024) + p, 16)] = sy - iy.astype(jnp.float32)
                wb[pl.ds(i32(2048) + p, 16)] = sz - iz.astype(jnp.float32)
                ax = (ix, ix + 1)
                b0 = iy * _P1
                by = (b0, b0 + _P1)
                c0 = iz * _P2
                cz = (c0, c0 + _P2)
                corner = 0
                for i in range(2):
                    for j in range(2):
                        for k in range(2):
                            h = ((ax[i] ^ by[j] ^ cz[k]) & _MASK) | lvl_off
                            idxb[pl.ds(i32(corner * _CH) + p, 16)] = h
                            corner += 1
                return c

            lax.fori_loop(jnp.int32(0), jnp.int32(_G), pass_a, 0)

            def fire(j, c):
                pltpu.async_copy(
                    tbl.at[idxb.at[pl.ds(j * i32(128), 128)]], rows.at[pl.ds(j * i32(128), 128)], sem)
                return c

            lax.fori_loop(jnp.int32(0), jnp.int32(_NROW), fire, 0)

            def drain(j, c):
                pltpu.make_async_copy(
                    tbl.at[idxb.at[pl.ds(j * i32(128), 128)]], rows.at[pl.ds(j * i32(128), 128)],
                    sem).wait()
                return c

            lax.fori_loop(jnp.int32(0), jnp.int32(_NROW), drain, 0)

            def pass_b(g, c, lvl=lvl):
                p = g * i32(16)
                w0 = wb[pl.ds(i32(0) + p, 16)]
                w1 = wb[pl.ds(i32(1024) + p, 16)]
                w2 = wb[pl.ds(i32(2048) + p, 16)]
                u1 = 1.0 - w1
                u2 = 1.0 - w2
                yz = (u1 * u2, u1 * w2, w1 * u2, w1 * w2)
                wx = (1.0 - w0, w0)
                acc0 = None
                acc1 = None
                corner = 0
                for i in range(2):
                    for jk in range(4):
                        wv = wx[i] * yz[jk]
                        gidx = (i32(corner * _CH) + p) + iota
                        f0 = plsc.load_gather(rows, [gidx, feat0])
                        f1 = plsc.load_gather(rows, [gidx, feat1])
                        if acc0 is None:
                            acc0 = wv * f0
                            acc1 = wv * f1
                        else:
                            acc0 = acc0 + wv * f0
                            acc1 = acc1 + wv * f1
                        corner += 1
                pidx = p + iota
                plsc.store_scatter(
                    ob, [pidx, jnp.full((16,), 2 * lvl, jnp.int32)], acc0)
                plsc.store_scatter(
                    ob, [pidx, jnp.full((16,), 2 * lvl + 1, jnp.int32)], acc1)
                return c

            lax.fori_loop(jnp.int32(0), jnp.int32(_G), pass_b, 0)

        pltpu.sync_copy(ob, out.at[pl.ds(cbase, _CH)])
        return carry

    lax.fori_loop(jnp.int32(0), jnp.int32(_NCHUNK), chunk_body, 0)


_enc = pl.kernel(
    _body,
    mesh=plsc.VectorSubcoreMesh(core_axis_name="c", subcore_axis_name="s"),
    out_type=jax.ShapeDtypeStruct((_N_POINTS, _N_LEVELS * _N_FEAT),
                                  jnp.float32),
    compiler_params=pltpu.CompilerParams(needs_layout_passes=False, use_tc_tiling_on_sc=False),
    scratch_types=[
        pltpu.VMEM((3 * _CH,), jnp.float32),
        pltpu.VMEM((3 * _CH,), jnp.float32),
        pltpu.VMEM((_CH * 8,), jnp.int32),
        pltpu.VMEM((_CH * 8, 2), jnp.float32),
        pltpu.VMEM((_CH, 32), jnp.float32),
        pltpu.SemaphoreType.DMA,
    ],
)


def kernel(x, tables):
    xt = jnp.transpose(x).reshape(-1)
    tbl = tables.reshape(_N_LEVELS * _HSIZE, _N_FEAT)
    return _enc(xt, tbl)


# speedup vs baseline: 2.0097x; 1.0008x over previous
import jax
import jax.numpy as jnp
import numpy as np
from jax import lax
from jax.experimental import pallas as pl
from jax.experimental.pallas import tpu as pltpu
from jax.experimental.pallas import tpu_sc as plsc

_N_POINTS = 524288
_N_LEVELS = 16
_N_FEAT = 2
_LOG2_HASH = 19
_HSIZE = 1 << _LOG2_HASH
_MASK = _HSIZE - 1
_GROWTH = np.exp((np.log(512.0) - np.log(16.0)) / (_N_LEVELS - 1))
_RESOLUTIONS = [int(np.ceil(16 * _GROWTH ** i)) for i in range(_N_LEVELS)]
_P1 = np.int32(2654435761 - (1 << 32))
_P2 = np.int32(805459861)

_NC, _NS = 2, 16
_NW = _NC * _NS
_PTS_W = _N_POINTS // _NW
_CH = 1024
_NCHUNK = _PTS_W // _CH
_G = _CH // 16
_NROW = _CH * 8 // 128


def _body(xt, tbl, out, xb, wb, idxb, rows, ob, sem):
    i32 = jnp.int32
    wid = lax.axis_index("s") * i32(_NC) + lax.axis_index("c")
    base = wid * i32(_PTS_W)
    iota = lax.iota(jnp.int32, 16)
    feat0 = jnp.zeros((16,), jnp.int32)
    feat1 = jnp.ones((16,), jnp.int32)

    def chunk_body(ch, carry):
        cbase = base + ch * i32(_CH)
        for d in range(3):
            pltpu.sync_copy(xt.at[pl.ds(i32(d * _N_POINTS) + cbase, _CH)],
                            xb.at[pl.ds(i32(d * _CH), _CH)])

        def clip_body(g, c):
            p = g * i32(16)
            for d in range(3):
                v = xb[pl.ds(i32(d * _CH) + p, 16)]
                xb[pl.ds(i32(d * _CH) + p, 16)] = jnp.minimum(
                    jnp.maximum(v, 0.0), 1.0 - 1e-6)
            return c

        lax.fori_loop(jnp.int32(0), jnp.int32(_G), clip_body, 0)

        for lvl in range(_N_LEVELS):
            res = jnp.float32(_RESOLUTIONS[lvl])
            lvl_off = jnp.int32(lvl << _LOG2_HASH)

            def pass_a(g, c, lvl_off=lvl_off, res=res):
                p = g * i32(16)
                r0 = g >> i32(3)
                q = (g & i32(7)) * i32(16)
                sx = xb[pl.ds(i32(0) + p, 16)] * res
                sy = xb[pl.ds(i32(1024) + p, 16)] * res
                sz = xb[pl.ds(i32(2048) + p, 16)] * res
                ix = sx.astype(jnp.int32)
                iy = sy.astype(jnp.int32)
                iz = sz.astype(jnp.int32)
                wb[pl.ds(i32(0) + p, 16)] = sx - ix.astype(jnp.float32)
                wb[pl.ds(i32(1024) + p, 16)] = sy - iy.astype(jnp.float32)
                wb[pl.ds(i32(2048) + p, 16)] = sz - iz.astype(jnp.float32)
                ax = (ix, ix + 1)
                b0 = iy * _P1
                by = (b0, b0 + _P1)
                c0 = iz * _P2
                cz = (c0, c0 + _P2)
                corner = 0
                for i in range(2):
                    for j in range(2):
                        for k in range(2):
                            h = ((ax[i] ^ by[j] ^ cz[k]) & _MASK) | lvl_off
                            idxb[pl.ds(i32(corner * _CH) + p, 16)] = h
                            corner += 1
                return c

            lax.fori_loop(jnp.int32(0), jnp.int32(_G), pass_a, 0)

            def fire(j, c):
                pltpu.async_copy(
                    tbl.at[idxb.at[pl.ds(j * i32(128), 128)]], rows.at[pl.ds(j * i32(128), 128)], sem)
                return c

            lax.fori_loop(jnp.int32(0), jnp.int32(_NROW), fire, 0)

            def drain(j, c):
                pltpu.make_async_copy(
                    tbl.at[idxb.at[pl.ds(j * i32(128), 128)]], rows.at[pl.ds(j * i32(128), 128)],
                    sem).wait()
                return c

            lax.fori_loop(jnp.int32(0), jnp.int32(_NROW), drain, 0)

            def pass_b(g, c, lvl=lvl):
                p = g * i32(16)
                w0 = wb[pl.ds(i32(0) + p, 16)]
                w1 = wb[pl.ds(i32(1024) + p, 16)]
                w2 = wb[pl.ds(i32(2048) + p, 16)]
                u1 = 1.0 - w1
                u2 = 1.0 - w2
                yz = (u1 * u2, u1 * w2, w1 * u2, w1 * w2)
                wx = (1.0 - w0, w0)
                acc0 = None
                acc1 = None
                corner = 0
                for i in range(2):
                    for jk in range(4):
                        wv = wx[i] * yz[jk]
                        gidx = (i32(corner * _CH) + p) + iota
                        f0 = plsc.load_gather(rows, [gidx, feat0])
                        f1 = plsc.load_gather(rows, [gidx, feat1])
                        if acc0 is None:
                            acc0 = wv * f0
                            acc1 = wv * f1
                        else:
                            acc0 = acc0 + wv * f0
                            acc1 = acc1 + wv * f1
                        corner += 1
                oidx = (p + iota) * i32(32) + i32(2 * lvl)
                plsc.store_scatter(ob, [oidx], acc0)
                plsc.store_scatter(ob, [oidx + i32(1)], acc1)
                return c

            lax.fori_loop(jnp.int32(0), jnp.int32(_G), pass_b, 0)

        pltpu.sync_copy(ob, out.at[pl.ds(cbase * i32(32), _CH * 32)])
        return carry

    lax.fori_loop(jnp.int32(0), jnp.int32(_NCHUNK), chunk_body, 0)


_enc = pl.kernel(
    _body,
    mesh=plsc.VectorSubcoreMesh(core_axis_name="c", subcore_axis_name="s"),
    out_type=jax.ShapeDtypeStruct((_N_POINTS * _N_LEVELS * _N_FEAT,),
                                  jnp.float32),
    compiler_params=pltpu.CompilerParams(needs_layout_passes=False, use_tc_tiling_on_sc=False),
    scratch_types=[
        pltpu.VMEM((3 * _CH,), jnp.float32),
        pltpu.VMEM((3 * _CH,), jnp.float32),
        pltpu.VMEM((_CH * 8,), jnp.int32),
        pltpu.VMEM((_CH * 8, 2), jnp.float32),
        pltpu.VMEM((_CH * 32,), jnp.float32),
        pltpu.SemaphoreType.DMA,
    ],
)


def kernel(x, tables):
    xt = jnp.transpose(x).reshape(-1)
    tbl = tables.reshape(_N_LEVELS * _HSIZE, _N_FEAT)
    return _enc(xt, tbl).reshape(_N_POINTS, _N_LEVELS * _N_FEAT)


# speedup vs baseline: 6.0029x; 2.9870x over previous
import jax
import jax.numpy as jnp
import numpy as np
from jax import lax
from jax.experimental import pallas as pl
from jax.experimental.pallas import tpu as pltpu
from jax.experimental.pallas import tpu_sc as plsc

_N_POINTS = 524288
_N_LEVELS = 16
_N_FEAT = 2
_LOG2_HASH = 19
_HSIZE = 1 << _LOG2_HASH
_MASK = _HSIZE - 1
_GROWTH = np.exp((np.log(512.0) - np.log(16.0)) / (_N_LEVELS - 1))
_RESOLUTIONS = [int(np.ceil(16 * _GROWTH ** i)) for i in range(_N_LEVELS)]
_P1 = np.int32(2654435761 - (1 << 32))
_P2 = np.int32(805459861)

_NC, _NS = 2, 16
_NW = _NC * _NS
_PTS_W = _N_POINTS // _NW
_CH = 1024
_NCHUNK = _PTS_W // _CH
_G = _CH // 16
_NROW = _CH * 8 // 128


def _body(xt, tbl, out, xb, wb, idxb, rows, ob, sem):
    i32 = jnp.int32
    wid = lax.axis_index("s") * i32(_NC) + lax.axis_index("c")
    base = wid * i32(_PTS_W)
    iota = lax.iota(jnp.int32, 16)

    def chunk_body(ch, carry):
        cbase = base + ch * i32(_CH)
        for d in range(3):
            pltpu.sync_copy(xt.at[pl.ds(i32(d * _N_POINTS) + cbase, _CH)],
                            xb.at[pl.ds(i32(d * _CH), _CH)])

        def clip_body(g, c):
            p = g * i32(16)
            for d in range(3):
                v = xb[pl.ds(i32(d * _CH) + p, 16)]
                xb[pl.ds(i32(d * _CH) + p, 16)] = jnp.minimum(
                    jnp.maximum(v, 0.0), 1.0 - 1e-6)
            return c

        lax.fori_loop(jnp.int32(0), jnp.int32(_G), clip_body, 0)

        for lvl in range(_N_LEVELS):
            res = jnp.float32(_RESOLUTIONS[lvl])
            lvl_off = jnp.int32(lvl << 20)

            def pass_a(g, c, lvl_off=lvl_off, res=res):
                p = g * i32(16)
                r0 = g >> i32(3)
                q = (g & i32(7)) * i32(16)
                sx = xb[pl.ds(i32(0) + p, 16)] * res
                sy = xb[pl.ds(i32(1024) + p, 16)] * res
                sz = xb[pl.ds(i32(2048) + p, 16)] * res
                ix = sx.astype(jnp.int32)
                iy = sy.astype(jnp.int32)
                iz = sz.astype(jnp.int32)
                wb[pl.ds(i32(0) + p, 16)] = sx - ix.astype(jnp.float32)
                wb[pl.ds(i32(1024) + p, 16)] = sy - iy.astype(jnp.float32)
                wb[pl.ds(i32(2048) + p, 16)] = sz - iz.astype(jnp.float32)
                ax = (ix, ix + 1)
                b0 = iy * _P1
                by = (b0, b0 + _P1)
                c0 = iz * _P2
                cz = (c0, c0 + _P2)
                corner = 0
                for i in range(2):
                    for j in range(2):
                        for k in range(2):
                            h = (ax[i] ^ by[j] ^ cz[k]) & _MASK
                            e0 = (((h >> i32(7)) << i32(8))
                                  | (h & i32(127)) | lvl_off)
                            idxb[pl.ds(i32(corner * _CH) + p, 16)] = e0
                            idxb[pl.ds(i32(8 * _CH + corner * _CH) + p,
                                       16)] = e0 + i32(128)
                            corner += 1
                return c

            lax.fori_loop(jnp.int32(0), jnp.int32(_G), pass_a, 0)

            def fire(j, c):
                sl = pl.ds(j * i32(128), 128)
                pltpu.async_copy(tbl.at[idxb.at[sl]], rows.at[sl], sem)
                return c

            lax.fori_loop(jnp.int32(0), jnp.int32(2 * _NROW), fire, 0)

            def drain(j, c):
                sl = pl.ds(j * i32(128), 128)
                pltpu.make_async_copy(tbl.at[idxb.at[sl]], rows.at[sl],
                                      sem).wait()
                return c

            lax.fori_loop(jnp.int32(0), jnp.int32(2 * _NROW), drain, 0)

            def pass_b(g, c, lvl=lvl):
                p = g * i32(16)
                w0 = wb[pl.ds(i32(0) + p, 16)]
                w1 = wb[pl.ds(i32(1024) + p, 16)]
                w2 = wb[pl.ds(i32(2048) + p, 16)]
                u1 = 1.0 - w1
                u2 = 1.0 - w2
                yz = (u1 * u2, u1 * w2, w1 * u2, w1 * w2)
                wx = (1.0 - w0, w0)
                acc0 = None
                acc1 = None
                corner = 0
                for i in range(2):
                    for jk in range(4):
                        wv = wx[i] * yz[jk]
                        f0 = rows[pl.ds(i32(corner * _CH) + p, 16)]
                        f1 = rows[pl.ds(i32(8 * _CH + corner * _CH) + p, 16)]
                        if acc0 is None:
                            acc0 = wv * f0
                            acc1 = wv * f1
                        else:
                            acc0 = acc0 + wv * f0
                            acc1 = acc1 + wv * f1
                        corner += 1
                b = g >> i32(3)
                q = (g & i32(7)) * i32(16)
                pos = b * i32(1024) + q
                f0c, f1c = 2 * lvl, 2 * lvl + 1
                ob[pl.ds(i32((f0c >> 3) * 8192 + (f0c & 7) * 128) + pos,
                         16)] = acc0
                ob[pl.ds(i32((f1c >> 3) * 8192 + (f1c & 7) * 128) + pos,
                         16)] = acc1
                return c

            lax.fori_loop(jnp.int32(0), jnp.int32(_G), pass_b, 0)

        c0 = cbase >> i32(7)
        for tr in range(4):
            pltpu.sync_copy(
                ob.at[pl.ds(i32(tr * 8192), 8192)],
                out.at[pl.ds(i32(tr * 4194304) + c0 * i32(1024), 8192)])
        return carry

    lax.fori_loop(jnp.int32(0), jnp.int32(_NCHUNK), chunk_body, 0)


_enc = pl.kernel(
    _body,
    mesh=plsc.VectorSubcoreMesh(core_axis_name="c", subcore_axis_name="s"),
    out_type=jax.ShapeDtypeStruct((_N_POINTS * _N_LEVELS * _N_FEAT,),
                                  jnp.float32),
    compiler_params=pltpu.CompilerParams(needs_layout_passes=False, use_tc_tiling_on_sc=False),
    scratch_types=[
        pltpu.VMEM((3 * _CH,), jnp.float32),
        pltpu.VMEM((3 * _CH,), jnp.float32),
        pltpu.VMEM((_CH * 16,), jnp.int32),
        pltpu.VMEM((_CH * 16,), jnp.float32),
        pltpu.VMEM((_CH * 32,), jnp.float32),
        pltpu.SemaphoreType.DMA,
    ],
)


def kernel(x, tables):
    xt = jnp.transpose(x).reshape(-1)
    tbl = (tables.reshape(_N_LEVELS, _HSIZE // 128, 128, _N_FEAT)
           .transpose(0, 1, 3, 2).reshape(-1))
    out = _enc(xt, tbl)
    return (out.reshape(4, _N_POINTS // 128, 8, 128)
            .transpose(1, 3, 0, 2).reshape(_N_POINTS, _N_LEVELS * _N_FEAT))


# speedup vs baseline: 6.0162x; 1.0022x over previous
import jax
import jax.numpy as jnp
import numpy as np
from jax import lax
from jax.experimental import pallas as pl
from jax.experimental.pallas import tpu as pltpu
from jax.experimental.pallas import tpu_sc as plsc

_N_POINTS = 524288
_N_LEVELS = 16
_N_FEAT = 2
_LOG2_HASH = 19
_HSIZE = 1 << _LOG2_HASH
_MASK = _HSIZE - 1
_GROWTH = np.exp((np.log(512.0) - np.log(16.0)) / (_N_LEVELS - 1))
_RESOLUTIONS = [int(np.ceil(16 * _GROWTH ** i)) for i in range(_N_LEVELS)]
_P1 = np.int32(2654435761 - (1 << 32))
_P2 = np.int32(805459861)

_NC, _NS = 2, 16
_NW = _NC * _NS
_PTS_W = _N_POINTS // _NW
_CH = 1024
_NCHUNK = _PTS_W // _CH
_G = _CH // 16
_NROW = _CH * 8 // 128
_DMA_N = 1024
_N_DMA = _CH * 16 // _DMA_N


def _body(xt, tbl, out, xb, wb, idxb, rows, ob, sem):
    i32 = jnp.int32
    wid = lax.axis_index("s") * i32(_NC) + lax.axis_index("c")
    base = wid * i32(_PTS_W)
    iota = lax.iota(jnp.int32, 16)

    def chunk_body(ch, carry):
        cbase = base + ch * i32(_CH)
        for d in range(3):
            pltpu.sync_copy(xt.at[pl.ds(i32(d * _N_POINTS) + cbase, _CH)],
                            xb.at[pl.ds(i32(d * _CH), _CH)])

        def clip_body(g, c):
            p = g * i32(16)
            for d in range(3):
                v = xb[pl.ds(i32(d * _CH) + p, 16)]
                xb[pl.ds(i32(d * _CH) + p, 16)] = jnp.minimum(
                    jnp.maximum(v, 0.0), 1.0 - 1e-6)
            return c

        lax.fori_loop(jnp.int32(0), jnp.int32(_G), clip_body, 0)

        for lvl in range(_N_LEVELS):
            res = jnp.float32(_RESOLUTIONS[lvl])
            lvl_off = jnp.int32(lvl << 20)

            def pass_a(g, c, lvl_off=lvl_off, res=res):
                p = g * i32(16)
                r0 = g >> i32(3)
                q = (g & i32(7)) * i32(16)
                sx = xb[pl.ds(i32(0) + p, 16)] * res
                sy = xb[pl.ds(i32(1024) + p, 16)] * res
                sz = xb[pl.ds(i32(2048) + p, 16)] * res
                ix = sx.astype(jnp.int32)
                iy = sy.astype(jnp.int32)
                iz = sz.astype(jnp.int32)
                wb[pl.ds(i32(0) + p, 16)] = sx - ix.astype(jnp.float32)
                wb[pl.ds(i32(1024) + p, 16)] = sy - iy.astype(jnp.float32)
                wb[pl.ds(i32(2048) + p, 16)] = sz - iz.astype(jnp.float32)
                ax = (ix, ix + 1)
                b0 = iy * _P1
                by = (b0, b0 + _P1)
                c0 = iz * _P2
                cz = (c0, c0 + _P2)
                corner = 0
                for i in range(2):
                    for j in range(2):
                        for k in range(2):
                            h = (ax[i] ^ by[j] ^ cz[k]) & _MASK
                            e0 = (((h >> i32(7)) << i32(8))
                                  | (h & i32(127)) | lvl_off)
                            idxb[pl.ds(i32(corner * _CH) + p, 16)] = e0
                            idxb[pl.ds(i32(8 * _CH + corner * _CH) + p,
                                       16)] = e0 + i32(128)
                            corner += 1
                return c

            lax.fori_loop(jnp.int32(0), jnp.int32(_G), pass_a, 0)

            def fire(j, c):
                sl = pl.ds(j * i32(_DMA_N), _DMA_N)
                pltpu.async_copy(tbl.at[idxb.at[sl]], rows.at[sl], sem)
                return c

            lax.fori_loop(jnp.int32(0), jnp.int32(_N_DMA), fire, 0)

            def drain(j, c):
                sl = pl.ds(j * i32(_DMA_N), _DMA_N)
                pltpu.make_async_copy(tbl.at[idxb.at[sl]], rows.at[sl],
                                      sem).wait()
                return c

            lax.fori_loop(jnp.int32(0), jnp.int32(_N_DMA), drain, 0)

            def pass_b(g, c, lvl=lvl):
                p = g * i32(16)
                w0 = wb[pl.ds(i32(0) + p, 16)]
                w1 = wb[pl.ds(i32(1024) + p, 16)]
                w2 = wb[pl.ds(i32(2048) + p, 16)]
                u1 = 1.0 - w1
                u2 = 1.0 - w2
                yz = (u1 * u2, u1 * w2, w1 * u2, w1 * w2)
                wx = (1.0 - w0, w0)
                acc0 = None
                acc1 = None
                corner = 0
                for i in range(2):
                    for jk in range(4):
                        wv = wx[i] * yz[jk]
                        f0 = rows[pl.ds(i32(corner * _CH) + p, 16)]
                        f1 = rows[pl.ds(i32(8 * _CH + corner * _CH) + p, 16)]
                        if acc0 is None:
                            acc0 = wv * f0
                            acc1 = wv * f1
                        else:
                            acc0 = acc0 + wv * f0
                            acc1 = acc1 + wv * f1
                        corner += 1
                b = g >> i32(3)
                q = (g & i32(7)) * i32(16)
                pos = b * i32(1024) + q
                f0c, f1c = 2 * lvl, 2 * lvl + 1
                ob[pl.ds(i32((f0c >> 3) * 8192 + (f0c & 7) * 128) + pos,
                         16)] = acc0
                ob[pl.ds(i32((f1c >> 3) * 8192 + (f1c & 7) * 128) + pos,
                         16)] = acc1
                return c

            lax.fori_loop(jnp.int32(0), jnp.int32(_G), pass_b, 0)

        c0 = cbase >> i32(7)
        for tr in range(4):
            pltpu.sync_copy(
                ob.at[pl.ds(i32(tr * 8192), 8192)],
                out.at[pl.ds(i32(tr * 4194304) + c0 * i32(1024), 8192)])
        return carry

    lax.fori_loop(jnp.int32(0), jnp.int32(_NCHUNK), chunk_body, 0)


_enc = pl.kernel(
    _body,
    mesh=plsc.VectorSubcoreMesh(core_axis_name="c", subcore_axis_name="s"),
    out_type=jax.ShapeDtypeStruct((_N_POINTS * _N_LEVELS * _N_FEAT,),
                                  jnp.float32),
    compiler_params=pltpu.CompilerParams(needs_layout_passes=False, use_tc_tiling_on_sc=False),
    scratch_types=[
        pltpu.VMEM((3 * _CH,), jnp.float32),
        pltpu.VMEM((3 * _CH,), jnp.float32),
        pltpu.VMEM((_CH * 16,), jnp.int32),
        pltpu.VMEM((_CH * 16,), jnp.float32),
        pltpu.VMEM((_CH * 32,), jnp.float32),
        pltpu.SemaphoreType.DMA,
    ],
)


def kernel(x, tables):
    xt = jnp.transpose(x).reshape(-1)
    tbl = (tables.reshape(_N_LEVELS, _HSIZE // 128, 128, _N_FEAT)
           .transpose(0, 1, 3, 2).reshape(-1))
    out = _enc(xt, tbl)
    return (out.reshape(4, _N_POINTS // 128, 8, 128)
            .transpose(1, 3, 0, 2).reshape(_N_POINTS, _N_LEVELS * _N_FEAT))


# speedup vs baseline: 6.7853x; 1.1278x over previous
import jax
import jax.numpy as jnp
import numpy as np
from jax import lax
from jax.experimental import pallas as pl
from jax.experimental.pallas import tpu as pltpu
from jax.experimental.pallas import tpu_sc as plsc

_N_POINTS = 524288
_N_LEVELS = 16
_N_FEAT = 2
_LOG2_HASH = 19
_HSIZE = 1 << _LOG2_HASH
_MASK = _HSIZE - 1
_GROWTH = np.exp((np.log(512.0) - np.log(16.0)) / (_N_LEVELS - 1))
_RESOLUTIONS = [int(np.ceil(16 * _GROWTH ** i)) for i in range(_N_LEVELS)]
_P1 = np.int32(2654435761 - (1 << 32))
_P2 = np.int32(805459861)

_NC, _NS = 2, 16
_NW = _NC * _NS
_PTS_W = _N_POINTS // _NW
_CH = 1024
_NCHUNK = _PTS_W // _CH
_G = _CH // 16
_EPC = _CH * 16
_DMA_N = 1024
_N_DMA = _EPC // _DMA_N


def _body(xt, tbl, out, xb, wb, idxb, rows, ob, sem0, sem1):
    i32 = jnp.int32
    sems = (sem0, sem1)
    wid = lax.axis_index("s") * i32(_NC) + lax.axis_index("c")
    base = wid * i32(_PTS_W)
    iota = lax.iota(jnp.int32, 16)

    def chunk_body(ch, carry):
        cbase = base + ch * i32(_CH)
        for d in range(3):
            pltpu.sync_copy(xt.at[pl.ds(i32(d * _N_POINTS) + cbase, _CH)],
                            xb.at[pl.ds(i32(d * _CH), _CH)])

        def clip_body(g, c):
            p = g * i32(16)
            for d in range(3):
                v = xb[pl.ds(i32(d * _CH) + p, 16)]
                xb[pl.ds(i32(d * _CH) + p, 16)] = jnp.minimum(
                    jnp.maximum(v, 0.0), 1.0 - 1e-6)
            return c

        lax.fori_loop(jnp.int32(0), jnp.int32(_G), clip_body, 0)

        def make_pass_a(lvl, par):
            res = jnp.float32(_RESOLUTIONS[lvl])
            lvl_off = jnp.int32(lvl << 20)
            ioff = par * _EPC
            woff = par * 3 * _CH

            def pass_a(g, c):
                p = g * i32(16)
                sx = xb[pl.ds(i32(0) + p, 16)] * res
                sy = xb[pl.ds(i32(_CH) + p, 16)] * res
                sz = xb[pl.ds(i32(2 * _CH) + p, 16)] * res
                ix = sx.astype(jnp.int32)
                iy = sy.astype(jnp.int32)
                iz = sz.astype(jnp.int32)
                wb[pl.ds(i32(woff) + p, 16)] = sx - ix.astype(jnp.float32)
                wb[pl.ds(i32(woff + _CH) + p, 16)] = (
                    sy - iy.astype(jnp.float32))
                wb[pl.ds(i32(woff + 2 * _CH) + p, 16)] = (
                    sz - iz.astype(jnp.float32))
                ax = (ix, ix + 1)
                b0 = iy * _P1
                by = (b0, b0 + _P1)
                c0 = iz * _P2
                cz = (c0, c0 + _P2)
                corner = 0
                for i in range(2):
                    for j in range(2):
                        for k in range(2):
                            h = (ax[i] ^ by[j] ^ cz[k]) & _MASK
                            e0 = (((h >> i32(7)) << i32(8))
                                  | (h & i32(127)) | lvl_off)
                            idxb[pl.ds(i32(ioff + corner * _CH) + p,
                                       16)] = e0
                            idxb[pl.ds(i32(ioff + (8 + corner) * _CH) + p,
                                       16)] = e0 + i32(128)
                            corner += 1
                return c

            return pass_a

        def fire(lvl, par):
            off = par * _EPC

            def fire_j(j, c):
                sl = pl.ds(i32(off) + j * i32(_DMA_N), _DMA_N)
                pltpu.async_copy(tbl.at[idxb.at[sl]], rows.at[sl], sems[par])
                return c

            lax.fori_loop(jnp.int32(0), jnp.int32(_N_DMA), fire_j, 0)

        def drain(lvl, par):
            off = par * _EPC

            def drain_j(j, c):
                sl = pl.ds(i32(off) + j * i32(_DMA_N), _DMA_N)
                pltpu.make_async_copy(tbl.at[idxb.at[sl]], rows.at[sl],
                                      sems[par]).wait()
                return c

            lax.fori_loop(jnp.int32(0), jnp.int32(_N_DMA), drain_j, 0)

        def make_pass_b(lvl, par):
            roff = par * _EPC
            woff = par * 3 * _CH

            def pass_b(g, c):
                p = g * i32(16)
                w0 = wb[pl.ds(i32(woff) + p, 16)]
                w1 = wb[pl.ds(i32(woff + _CH) + p, 16)]
                w2 = wb[pl.ds(i32(woff + 2 * _CH) + p, 16)]
                u1 = 1.0 - w1
                u2 = 1.0 - w2
                yz = (u1 * u2, u1 * w2, w1 * u2, w1 * w2)
                wx = (1.0 - w0, w0)
                acc0 = None
                acc1 = None
                corner = 0
                for i in range(2):
                    for jk in range(4):
                        wv = wx[i] * yz[jk]
                        f0 = rows[pl.ds(i32(roff + corner * _CH) + p, 16)]
                        f1 = rows[pl.ds(i32(roff + (8 + corner) * _CH) + p,
                                        16)]
                        if acc0 is None:
                            acc0 = wv * f0
                            acc1 = wv * f1
                        else:
                            acc0 = acc0 + wv * f0
                            acc1 = acc1 + wv * f1
                        corner += 1
                b = g >> i32(3)
                q = (g & i32(7)) * i32(16)
                pos = b * i32(1024) + q
                f0c, f1c = 2 * lvl, 2 * lvl + 1
                ob[pl.ds(i32((f0c >> 3) * 8192 + (f0c & 7) * 128) + pos,
                         16)] = acc0
                ob[pl.ds(i32((f1c >> 3) * 8192 + (f1c & 7) * 128) + pos,
                         16)] = acc1
                return c

            return pass_b

        lax.fori_loop(jnp.int32(0), jnp.int32(_G), make_pass_a(0, 0), 0)
        fire(0, 0)
        for lvl in range(_N_LEVELS):
            par = lvl & 1
            if lvl + 1 < _N_LEVELS:
                lax.fori_loop(jnp.int32(0), jnp.int32(_G),
                              make_pass_a(lvl + 1, par ^ 1), 0)
                fire(lvl + 1, par ^ 1)
            drain(lvl, par)
            lax.fori_loop(jnp.int32(0), jnp.int32(_G),
                          make_pass_b(lvl, par), 0)

        c0 = cbase >> i32(7)
        for tr in range(4):
            pltpu.sync_copy(
                ob.at[pl.ds(i32(tr * 8192), 8192)],
                out.at[pl.ds(i32(tr * 4194304) + c0 * i32(1024), 8192)])
        return carry

    lax.fori_loop(jnp.int32(0), jnp.int32(_NCHUNK), chunk_body, 0)


_enc = pl.kernel(
    _body,
    mesh=plsc.VectorSubcoreMesh(core_axis_name="c", subcore_axis_name="s"),
    out_type=jax.ShapeDtypeStruct((_N_POINTS * _N_LEVELS * _N_FEAT,),
                                  jnp.float32),
    compiler_params=pltpu.CompilerParams(needs_layout_passes=False,
                                         use_tc_tiling_on_sc=False),
    scratch_types=[
        pltpu.VMEM((3 * _CH,), jnp.float32),
        pltpu.VMEM((2 * 3 * _CH,), jnp.float32),
        pltpu.VMEM((2 * _EPC,), jnp.int32),
        pltpu.VMEM((2 * _EPC,), jnp.float32),
        pltpu.VMEM((_CH * 32,), jnp.float32),
        pltpu.SemaphoreType.DMA,
        pltpu.SemaphoreType.DMA,
    ],
)


def kernel(x, tables):
    xt = jnp.transpose(x).reshape(-1)
    tbl = (tables.reshape(_N_LEVELS, _HSIZE // 128, 128, _N_FEAT)
           .transpose(0, 1, 3, 2).reshape(-1))
    out = _enc(xt, tbl)
    return (out.reshape(4, _N_POINTS // 128, 8, 128)
            .transpose(1, 3, 0, 2).reshape(_N_POINTS, _N_LEVELS * _N_FEAT))
